# baseline (speedup 1.0000x reference)
import functools

import jax
import jax.numpy as jnp
from jax import lax
from jax.experimental import pallas as pl
from jax.experimental.pallas import tpu as pltpu
from jax.experimental.pallas import tpu_sc as plsc

N = 10000
E = 320000
D = 128

NC = 2
NS = 16
NW = NC * NS

CHUNK = 128
CH = 80
NHALF = 2
HCH = CH // NHALF
EW = CH * CHUNK
E_PAD = NW * EW

N_PAD = 10240
RPT = N_PAD // NS
RC = RPT // CHUNK


def _make_sc_spmm(with_deg):
    def body(feats, e2, zrows, zdeg, *refs):
        if with_deg:
            (agg_out, deg_out, idx_b, rows, ones_v, deg_v,
             agg_sh, deg_sh, sg0, sg1, ss0, ss1, sd) = refs
        else:
            agg_out, idx_b, rows, agg_sh, sg0, sg1, ss0, ss1, sd = refs
        sem_g = (sg0, sg1)
        c = lax.axis_index("c")
        s = lax.axis_index("s")
        w = c * NS + s

        if with_deg:
            for k in range(CHUNK // 16):
                ones_v[pl.ds(k * 16, 16)] = jnp.ones((16,), jnp.float32)

        pltpu.sync_copy(zrows, rows.at[0])
        for k in range(RC):
            pltpu.async_copy(rows.at[0],
                             agg_sh.at[pl.ds(s * RPT + k * CHUNK, CHUNK)],
                             sem_g[0])
        if with_deg:
            pltpu.sync_copy(zdeg, deg_v)
            pltpu.async_copy(deg_v, deg_sh.at[pl.ds(s * RPT, RPT)], sem_g[1])
        for k in range(RC):
            pltpu.make_async_copy(
                rows.at[0], agg_sh.at[pl.ds(s * RPT + k * CHUNK, CHUNK)],
                sem_g[0]).wait()
        if with_deg:
            pltpu.make_async_copy(
                deg_v, deg_sh.at[pl.ds(s * RPT, RPT)], sem_g[1]).wait()
        plsc.subcore_barrier()

        sem_s = (ss0, ss1)

        def arm(j, sl, first):
            if not first:
                pltpu.make_async_copy(rows.at[sl], agg_sh.at[idx_b.at[j, 1]],
                                      sem_s[sl]).wait()
            pltpu.async_copy(feats.at[idx_b.at[j, 0]], rows.at[sl],
                             sem_g[sl])

        def fire(j, sl):
            pltpu.make_async_copy(feats.at[idx_b.at[j, 0]], rows.at[sl],
                                  sem_g[sl]).wait()
            pltpu.async_copy(rows.at[sl], agg_sh.at[idx_b.at[j, 1]],
                             sem_s[sl], add=True)
            if with_deg:
                pltpu.async_copy(ones_v, deg_sh.at[idx_b.at[j, 1]], sd,
                                 add=True)

        for half in range(NHALF):
            pltpu.sync_copy(e2.at[pl.ds(w * CH + half * HCH, HCH)], idx_b)
            arm(0, 0, True)
            arm(1, 1, True)

            @pl.loop(0, HCH // 2 - 1)
            def _(i):
                j0 = i * 2
                fire(j0, 0)
                arm(j0 + 2, 0, False)
                fire(j0 + 1, 1)
                arm(j0 + 3, 1, False)

            fire(HCH - 2, 0)
            fire(HCH - 1, 1)
            for sl in range(2):
                pltpu.make_async_copy(rows.at[sl],
                                      agg_sh.at[idx_b.at[sl, 1]],
                                      sem_s[sl]).wait()
            if with_deg:
                @pl.loop(0, HCH)
                def _(j):
                    pltpu.make_async_copy(ones_v, deg_sh.at[idx_b.at[j, 1]],
                                          sd).wait()

        plsc.subcore_barrier()

        for k in range(RC):
            sl = k % 2
            if k >= 2:
                pltpu.make_async_copy(
                    rows.at[sl],
                    agg_out.at[c, pl.ds(s * RPT + (k - 2) * CHUNK, CHUNK)],
                    sem_g[sl]).wait()
            r0 = s * RPT + k * CHUNK
            pltpu.sync_copy(agg_sh.at[pl.ds(r0, CHUNK)], rows.at[sl])
            pltpu.async_copy(rows.at[sl], agg_out.at[c, pl.ds(r0, CHUNK)],
                             sem_g[sl])
        if with_deg:
            pltpu.sync_copy(deg_sh.at[pl.ds(s * RPT, RPT)], deg_v)
            pltpu.async_copy(deg_v, deg_out.at[c, pl.ds(s * RPT, RPT)], sd)
        for k in range(max(0, RC - 2), RC):
            sl = k % 2
            pltpu.make_async_copy(
                rows.at[sl], agg_out.at[c, pl.ds(s * RPT + k * CHUNK, CHUNK)],
                sem_g[sl]).wait()
        if with_deg:
            pltpu.make_async_copy(
                deg_v, deg_out.at[c, pl.ds(s * RPT, RPT)], sd).wait()

    if with_deg:
        out_type = (
            jax.ShapeDtypeStruct((NC, N_PAD, D), jnp.float32),
            jax.ShapeDtypeStruct((NC, N_PAD), jnp.float32),
        )
    else:
        out_type = jax.ShapeDtypeStruct((NC, N_PAD, D), jnp.float32)
    scratch = [
        pltpu.VMEM((HCH, 2, CHUNK), jnp.int32),
        pltpu.VMEM((2, CHUNK, D), jnp.float32),
    ]
    if with_deg:
        scratch += [
            pltpu.VMEM((CHUNK,), jnp.float32),
            pltpu.VMEM((RPT,), jnp.float32),
        ]
    scratch += [pltpu.VMEM_SHARED((N_PAD, D), jnp.float32)]
    if with_deg:
        scratch += [pltpu.VMEM_SHARED((N_PAD,), jnp.float32)]
    scratch += [pltpu.SemaphoreType.DMA] * 5
    return pl.kernel(
        body,
        out_type=out_type,
        mesh=plsc.VectorSubcoreMesh(
            core_axis_name="c", subcore_axis_name="s",
            num_cores=NC, num_subcores=NS),
        scratch_types=scratch,
    )


_sc_spmm_deg = _make_sc_spmm(True)
_sc_spmm = _make_sc_spmm(False)


BLK = 2000


def _dense_body(elu, a_ref, d_ref, x_ref, wl_ref, b_ref, wr_ref, o_ref):
    deg = d_ref[0, :, :] + d_ref[1, :, :]
    rdeg = 1.0 / jnp.maximum(deg, 1.0)
    agg = (a_ref[0, :, :] + a_ref[1, :, :]) * rdeg
    h = jnp.dot(agg, wl_ref[...], preferred_element_type=jnp.float32)
    h = h + b_ref[...]
    h = h + jnp.dot(x_ref[...], wr_ref[...], preferred_element_type=jnp.float32)
    if elu:
        h = jnp.where(h > 0.0, h, jnp.exp(h) - 1.0)
    o_ref[...] = h


def _dense(agg_p, deg_p, x, W_l, b_l, W_r, elu):
    return pl.pallas_call(
        functools.partial(_dense_body, elu),
        grid=(N // BLK,),
        in_specs=[
            pl.BlockSpec((NC, BLK, D), lambda i: (0, i, 0)),
            pl.BlockSpec((NC, BLK, 1), lambda i: (0, i, 0)),
            pl.BlockSpec((BLK, D), lambda i: (i, 0)),
            pl.BlockSpec((D, D), lambda i: (0, 0)),
            pl.BlockSpec((1, D), lambda i: (0, 0)),
            pl.BlockSpec((D, D), lambda i: (0, 0)),
        ],
        out_specs=pl.BlockSpec((BLK, D), lambda i: (i, 0)),
        out_shape=jax.ShapeDtypeStruct((N, D), jnp.float32),
    )(agg_p, deg_p, x, W_l, b_l, W_r)


def kernel(x, edge_index, W_l0, b_l0, W_r0, W_l1, b_l1, W_r1):
    src = edge_index[0].astype(jnp.int32)
    dst = edge_index[1].astype(jnp.int32)
    pad = E_PAD - E
    lanes = jnp.arange(pad, dtype=jnp.int32) % NW
    src_p = jnp.concatenate([src, lanes]).reshape(NW * CH, 1, CHUNK)
    dst_p = jnp.concatenate([dst, N + lanes]).reshape(NW * CH, 1, CHUNK)
    e2 = jnp.concatenate([src_p, dst_p], axis=1)
    zrows = jnp.zeros((CHUNK, D), jnp.float32)
    zdeg = jnp.zeros((RPT,), jnp.float32)

    agg_p, deg_p = _sc_spmm_deg(x, e2, zrows, zdeg)
    deg3 = deg_p.reshape(NC, N_PAD, 1)
    h = _dense(agg_p, deg3, x, W_l0, b_l0.reshape(1, D), W_r0, True)
    agg2_p = _sc_spmm(h, e2, zrows, zdeg)
    out = _dense(agg2_p, deg3, h, W_l1, b_l1.reshape(1, D), W_r1, False)
    return out


# speedup vs baseline: 1.0062x; 1.0062x over previous
import functools

import jax
import jax.numpy as jnp
from jax import lax
from jax.experimental import pallas as pl
from jax.experimental.pallas import tpu as pltpu
from jax.experimental.pallas import tpu_sc as plsc

N = 10000
E = 320000
D = 128

NC = 2
NS = 16
NW = NC * NS

CHUNK = 128
CH = 80
NHALF = 2
HCH = CH // NHALF
EW = CH * CHUNK
E_PAD = NW * EW

N_PAD = 10240
RPT = N_PAD // NS
RC = RPT // CHUNK


def _make_sc_spmm(with_deg):
    def body(feats, e2, zrows, zdeg, *refs):
        if with_deg:
            (agg_out, deg_out, idx_b, rows, ones_v, deg_v,
             agg_sh, deg_sh, sg0, sg1, sd) = refs
        else:
            agg_out, idx_b, rows, agg_sh, sg0, sg1, sd = refs
        sem_g = (sg0, sg1)
        c = lax.axis_index("c")
        s = lax.axis_index("s")
        w = c * NS + s

        if with_deg:
            for k in range(CHUNK // 16):
                ones_v[pl.ds(k * 16, 16)] = jnp.ones((16,), jnp.float32)

        pltpu.sync_copy(zrows, rows.at[0])
        for k in range(RC):
            pltpu.async_copy(rows.at[0],
                             agg_sh.at[pl.ds(s * RPT + k * CHUNK, CHUNK)],
                             sem_g[0])
        if with_deg:
            pltpu.sync_copy(zdeg, deg_v)
            pltpu.async_copy(deg_v, deg_sh.at[pl.ds(s * RPT, RPT)], sem_g[1])
        for k in range(RC):
            pltpu.make_async_copy(
                rows.at[0], agg_sh.at[pl.ds(s * RPT + k * CHUNK, CHUNK)],
                sem_g[0]).wait()
        if with_deg:
            pltpu.make_async_copy(
                deg_v, deg_sh.at[pl.ds(s * RPT, RPT)], sem_g[1]).wait()
        plsc.subcore_barrier()

        def arm(j, sl):
            pltpu.async_copy(feats.at[idx_b.at[j, 0]], rows.at[sl],
                             sem_g[sl])

        def fire(j, sl):
            pltpu.make_async_copy(feats.at[idx_b.at[j, 0]], rows.at[sl],
                                  sem_g[sl]).wait()
            pltpu.sync_copy(rows.at[sl], agg_sh.at[idx_b.at[j, 1]], add=True)
            if with_deg:
                pltpu.async_copy(ones_v, deg_sh.at[idx_b.at[j, 1]], sd,
                                 add=True)

        for half in range(NHALF):
            pltpu.sync_copy(e2.at[pl.ds(w * CH + half * HCH, HCH)], idx_b)
            arm(0, 0)
            arm(1, 1)

            @pl.loop(0, HCH // 2 - 1)
            def _(i):
                j0 = i * 2
                fire(j0, 0)
                arm(j0 + 2, 0)
                fire(j0 + 1, 1)
                arm(j0 + 3, 1)

            fire(HCH - 2, 0)
            fire(HCH - 1, 1)
            if with_deg:
                @pl.loop(0, HCH)
                def _(j):
                    pltpu.make_async_copy(ones_v, deg_sh.at[idx_b.at[j, 1]],
                                          sd).wait()

        plsc.subcore_barrier()

        for k in range(RC):
            sl = k % 2
            if k >= 2:
                pltpu.make_async_copy(
                    rows.at[sl],
                    agg_out.at[c, pl.ds(s * RPT + (k - 2) * CHUNK, CHUNK)],
                    sem_g[sl]).wait()
            r0 = s * RPT + k * CHUNK
            pltpu.sync_copy(agg_sh.at[pl.ds(r0, CHUNK)], rows.at[sl])
            pltpu.async_copy(rows.at[sl], agg_out.at[c, pl.ds(r0, CHUNK)],
                             sem_g[sl])
        if with_deg:
            pltpu.sync_copy(deg_sh.at[pl.ds(s * RPT, RPT)], deg_v)
            pltpu.async_copy(deg_v, deg_out.at[c, pl.ds(s * RPT, RPT)], sd)
        for k in range(max(0, RC - 2), RC):
            sl = k % 2
            pltpu.make_async_copy(
                rows.at[sl], agg_out.at[c, pl.ds(s * RPT + k * CHUNK, CHUNK)],
                sem_g[sl]).wait()
        if with_deg:
            pltpu.make_async_copy(
                deg_v, deg_out.at[c, pl.ds(s * RPT, RPT)], sd).wait()

    if with_deg:
        out_type = (
            jax.ShapeDtypeStruct((NC, N_PAD, D), jnp.float32),
            jax.ShapeDtypeStruct((NC, N_PAD), jnp.float32),
        )
    else:
        out_type = jax.ShapeDtypeStruct((NC, N_PAD, D), jnp.float32)
    scratch = [
        pltpu.VMEM((HCH, 2, CHUNK), jnp.int32),
        pltpu.VMEM((2, CHUNK, D), jnp.float32),
    ]
    if with_deg:
        scratch += [
            pltpu.VMEM((CHUNK,), jnp.float32),
            pltpu.VMEM((RPT,), jnp.float32),
        ]
    scratch += [pltpu.VMEM_SHARED((N_PAD, D), jnp.float32)]
    if with_deg:
        scratch += [pltpu.VMEM_SHARED((N_PAD,), jnp.float32)]
    scratch += [pltpu.SemaphoreType.DMA] * 3
    return pl.kernel(
        body,
        out_type=out_type,
        mesh=plsc.VectorSubcoreMesh(
            core_axis_name="c", subcore_axis_name="s",
            num_cores=NC, num_subcores=NS),
        scratch_types=scratch,
    )


_sc_spmm_deg = _make_sc_spmm(True)
_sc_spmm = _make_sc_spmm(False)


BLK = 2000


def _dense_body(elu, a_ref, d_ref, x_ref, wl_ref, b_ref, wr_ref, o_ref):
    deg = d_ref[0, :, :] + d_ref[1, :, :]
    rdeg = 1.0 / jnp.maximum(deg, 1.0)
    agg = (a_ref[0, :, :] + a_ref[1, :, :]) * rdeg
    h = jnp.dot(agg, wl_ref[...], preferred_element_type=jnp.float32)
    h = h + b_ref[...]
    h = h + jnp.dot(x_ref[...], wr_ref[...], preferred_element_type=jnp.float32)
    if elu:
        h = jnp.where(h > 0.0, h, jnp.exp(h) - 1.0)
    o_ref[...] = h


def _dense(agg_p, deg_p, x, W_l, b_l, W_r, elu):
    return pl.pallas_call(
        functools.partial(_dense_body, elu),
        grid=(N // BLK,),
        in_specs=[
            pl.BlockSpec((NC, BLK, D), lambda i: (0, i, 0)),
            pl.BlockSpec((NC, BLK, 1), lambda i: (0, i, 0)),
            pl.BlockSpec((BLK, D), lambda i: (i, 0)),
            pl.BlockSpec((D, D), lambda i: (0, 0)),
            pl.BlockSpec((1, D), lambda i: (0, 0)),
            pl.BlockSpec((D, D), lambda i: (0, 0)),
        ],
        out_specs=pl.BlockSpec((BLK, D), lambda i: (i, 0)),
        out_shape=jax.ShapeDtypeStruct((N, D), jnp.float32),
    )(agg_p, deg_p, x, W_l, b_l, W_r)


def kernel(x, edge_index, W_l0, b_l0, W_r0, W_l1, b_l1, W_r1):
    src = edge_index[0].astype(jnp.int32)
    dst = edge_index[1].astype(jnp.int32)
    pad = E_PAD - E
    lanes = jnp.arange(pad, dtype=jnp.int32) % NW
    src_p = jnp.concatenate([src, lanes]).reshape(NW * CH, 1, CHUNK)
    dst_p = jnp.concatenate([dst, N + lanes]).reshape(NW * CH, 1, CHUNK)
    e2 = jnp.concatenate([src_p, dst_p], axis=1)
    zrows = jnp.zeros((CHUNK, D), jnp.float32)
    zdeg = jnp.zeros((RPT,), jnp.float32)

    agg_p, deg_p = _sc_spmm_deg(x, e2, zrows, zdeg)
    deg3 = deg_p.reshape(NC, N_PAD, 1)
    h = _dense(agg_p, deg3, x, W_l0, b_l0.reshape(1, D), W_r0, True)
    agg2_p = _sc_spmm(h, e2, zrows, zdeg)
    out = _dense(agg2_p, deg3, h, W_l1, b_l1.reshape(1, D), W_r1, False)
    return out
